# baseline (speedup 1.0000x reference)
import functools

import jax
import jax.numpy as jnp
from jax import lax
from jax.experimental import pallas as pl
from jax.experimental.pallas import tpu as pltpu
from jax.experimental.pallas import tpu_sc as plsc

_B, _C, _H, _W = 16, 384, 56, 56
_HW = _H * _W
_C2 = 2 * _C
_N = _B * _HW
_EPS = 1e-5
_HWB = 392
_LANES = 16
_NJ = _C // _LANES
_CH = 16
_HWT = _HW // 2
_NCH = _HWT // _CH
_NB = 2


def _stats_body(x1_ref, x2_ref, s1_ref, q1_ref, s2_ref, q2_ref):
    h = pl.program_id(1)
    x1 = x1_ref[0]
    x2 = x2_ref[0]
    s1 = jnp.sum(x1, axis=0, keepdims=True).reshape(1, 1, _C)
    q1 = jnp.sum(x1 * x1, axis=0, keepdims=True).reshape(1, 1, _C)
    s2 = jnp.sum(x2, axis=0, keepdims=True).reshape(1, 1, _C)
    q2 = jnp.sum(x2 * x2, axis=0, keepdims=True).reshape(1, 1, _C)

    @pl.when(h == 0)
    def _():
        s1_ref[...] = s1
        q1_ref[...] = q1
        s2_ref[...] = s2
        q2_ref[...] = q2

    @pl.when(h != 0)
    def _():
        s1_ref[...] += s1
        q1_ref[...] += q1
        s2_ref[...] += s2
        q2_ref[...] += q2


def _stats(x1t, x2t):
    out = jax.ShapeDtypeStruct((_B, 1, _C), jnp.float32)
    return pl.pallas_call(
        _stats_body,
        grid=(_B, _HW // _HWB),
        in_specs=[
            pl.BlockSpec((1, _HWB, _C), lambda b, h: (b, h, 0)),
            pl.BlockSpec((1, _HWB, _C), lambda b, h: (b, h, 0)),
        ],
        out_specs=[pl.BlockSpec((1, 1, _C), lambda b, h: (b, 0, 0))] * 4,
        out_shape=[out] * 4,
    )(x1t, x2t)


def _row_to_col(row):
    n = row.shape[1]
    ci = lax.broadcasted_iota(jnp.int32, (n, n), 0)
    cj = lax.broadcasted_iota(jnp.int32, (n, n), 1)
    return jnp.sum(jnp.where(ci == cj, row, 0.0), axis=1, keepdims=True)


def _select_body(s1_ref, q1_ref, s2_ref, q2_ref, g_ref, b_ref,
                 idx_ref, sc_ref, sh_ref):
    ci = lax.broadcasted_iota(jnp.int32, (_C2, _C2), 0)
    cj = lax.broadcasted_iota(jnp.int32, (_C2, _C2), 1)
    jrow = lax.broadcasted_iota(jnp.int32, (_C2, _C), 1)
    cval = lax.broadcasted_iota(jnp.int32, (_C2, _C), 0)

    def per_b(b, carry):
        ssum, qsum = carry
        vrow = jnp.concatenate(
            [s1_ref[pl.ds(b, 1), 0, :], s2_ref[pl.ds(b, 1), 0, :]], axis=1)
        qrow = jnp.concatenate(
            [q1_ref[pl.ds(b, 1), 0, :], q2_ref[pl.ds(b, 1), 0, :]], axis=1)
        vcol = _row_to_col(vrow)
        qcol = _row_to_col(qrow)
        beats = (vrow > vcol) | ((vrow == vcol) & (cj < ci))
        rank = jnp.sum(jnp.where(beats, 1, 0), axis=1, keepdims=True)
        iseq = rank == jrow
        onehot = jnp.where(iseq, 1.0, 0.0)
        ssum = ssum + jnp.sum(onehot * vcol, axis=0, keepdims=True)
        qsum = qsum + jnp.sum(onehot * qcol, axis=0, keepdims=True)
        idxrow = jnp.sum(jnp.where(iseq, cval, 0), axis=0, keepdims=True)
        idx_ref[pl.ds(b, 1), :] = idxrow
        return ssum, qsum

    zero = jnp.zeros((1, _C), jnp.float32)
    ssum, qsum = lax.fori_loop(0, _B, per_b, (zero, zero))
    mean = ssum * (1.0 / _N)
    var = qsum * (1.0 / _N) - mean * mean
    scale = g_ref[...] * lax.rsqrt(var + _EPS)
    sc_ref[...] = scale
    sh_ref[...] = b_ref[...] - mean * scale


def _select(s1, q1, s2, q2, gamma, beta):
    mat = pl.BlockSpec((_B, 1, _C), lambda: (0, 0, 0))
    vec = pl.BlockSpec((1, _C), lambda: (0, 0))
    return pl.pallas_call(
        _select_body,
        in_specs=[mat, mat, mat, mat, vec, vec],
        out_specs=[
            pl.BlockSpec((_B, _C), lambda: (0, 0)),
            vec,
            vec,
        ],
        out_shape=[
            jax.ShapeDtypeStruct((_B, _C), jnp.int32),
            jax.ShapeDtypeStruct((1, _C), jnp.float32),
            jax.ShapeDtypeStruct((1, _C), jnp.float32),
        ],
    )(s1, q1, s2, q2, gamma, beta)


def _gather_body(x1_ref, x2_ref, idx_ref, sc_ref, sh_ref, out_ref,
                 idx_v, sc_v, sh_v, ibuf, obuf, g1sem, g2sem, ssem):
    cid = lax.axis_index("c")
    sid = lax.axis_index("s")
    b = sid
    hw0 = cid * _HWT

    pltpu.sync_copy(idx_ref.at[b], idx_v)
    pltpu.sync_copy(sc_ref.at[0], sc_v)
    pltpu.sync_copy(sh_ref.at[0], sh_v)

    def issue_in(k, slot):
        pltpu.async_copy(x1_ref.at[b, pl.ds(hw0 + k * _CH, _CH), :],
                         ibuf.at[pl.ds(slot * _CH, _CH), pl.ds(0, _C)],
                         g1sem.at[slot])
        pltpu.async_copy(x2_ref.at[b, pl.ds(hw0 + k * _CH, _CH), :],
                         ibuf.at[pl.ds(slot * _CH, _CH), pl.ds(_C, _C)],
                         g2sem.at[slot])

    def compute_chunk(slot):
        for j in range(_NJ):
            idxv = idx_v[pl.ds(j * _LANES, _LANES)]
            scv = sc_v[pl.ds(j * _LANES, _LANES)]
            shv = sh_v[pl.ds(j * _LANES, _LANES)]

            def row_body(r, carry, slot=slot, j=j, idxv=idxv, scv=scv,
                         shv=shv):
                rowv = jnp.full((_LANES,), slot * _CH + r, jnp.int32)
                g = plsc.load_gather(ibuf, [rowv, idxv])
                y = jnp.maximum(g * scv + shv, 0.0)
                obuf[slot, r, pl.ds(j * _LANES, _LANES)] = y
                return carry

            lax.fori_loop(0, _CH, row_body, 0, unroll=4)

    for s in range(_NB):
        issue_in(s, s)

    def outer(t, carry):
        for s in range(_NB):
            k = t * _NB + s
            pltpu.make_async_copy(x1_ref.at[0, pl.ds(0, _CH), :],
                                  ibuf.at[pl.ds(s * _CH, _CH), pl.ds(0, _C)],
                                  g1sem.at[s]).wait()
            pltpu.make_async_copy(x2_ref.at[0, pl.ds(0, _CH), :],
                                  ibuf.at[pl.ds(s * _CH, _CH), pl.ds(_C, _C)],
                                  g2sem.at[s]).wait()

            @pl.when(t > 0)
            def _():
                pltpu.make_async_copy(obuf.at[s],
                                      out_ref.at[0, pl.ds(0, _CH), :],
                                      ssem.at[s]).wait()

            compute_chunk(s)
            pltpu.async_copy(obuf.at[s],
                             out_ref.at[b, pl.ds(hw0 + k * _CH, _CH), :],
                             ssem.at[s])

            @pl.when(k + _NB < _NCH)
            def _():
                issue_in(k + _NB, s)
        return carry

    lax.fori_loop(0, _NCH // _NB, outer, 0)

    for s in range(_NB):
        pltpu.make_async_copy(obuf.at[s], out_ref.at[0, pl.ds(0, _CH), :],
                              ssem.at[s]).wait()


def _gather(x1t, x2t, idx, scale, shift):
    mesh = plsc.VectorSubcoreMesh(core_axis_name="c", subcore_axis_name="s")
    fn = pl.kernel(
        _gather_body,
        out_type=jax.ShapeDtypeStruct((_B, _HW, _C), jnp.float32),
        mesh=mesh,
        compiler_params=pltpu.CompilerParams(needs_layout_passes=False),
        scratch_types=[
            pltpu.VMEM((_C,), jnp.int32),
            pltpu.VMEM((_C,), jnp.float32),
            pltpu.VMEM((_C,), jnp.float32),
            pltpu.VMEM((_NB * _CH, _C2), jnp.float32),
            pltpu.VMEM((_NB, _CH, _C), jnp.float32),
            pltpu.SemaphoreType.DMA((_NB,)),
            pltpu.SemaphoreType.DMA((_NB,)),
            pltpu.SemaphoreType.DMA((_NB,)),
        ],
    )
    return fn(x1t, x2t, idx, scale, shift)


def kernel(x1, x2, gamma, beta):
    x1t = x1.reshape(_B, _C, _HW).transpose(0, 2, 1)
    x2t = x2.reshape(_B, _C, _HW).transpose(0, 2, 1)
    s1, q1, s2, q2 = _stats(x1t, x2t)
    idx, scale, shift = _select(s1, q1, s2, q2,
                                gamma.reshape(1, _C), beta.reshape(1, _C))
    out = _gather(x1t, x2t, idx, scale, shift)
    return out.transpose(0, 2, 1).reshape(_B, _C, _H, _W)


# speedup vs baseline: 2.0074x; 2.0074x over previous
import functools

import jax
import jax.numpy as jnp
from jax import lax
from jax.experimental import pallas as pl
from jax.experimental.pallas import tpu as pltpu
from jax.experimental.pallas import tpu_sc as plsc

_B, _C, _H, _W = 16, 384, 56, 56
_HW = _H * _W
_C2 = 2 * _C
_N = _B * _HW
_EPS = 1e-5
_HWB = 784
_LANES = 16
_NJ = _C // _LANES
_CH = 8
_HWT = _HW // 2
_NCH = _HWT // _CH
_NB = 4


def _stats_body(x1_ref, x2_ref, s1_ref, q1_ref, s2_ref, q2_ref):
    h = pl.program_id(1)
    x1 = x1_ref[0]
    x2 = x2_ref[0]
    s1 = jnp.sum(x1, axis=0, keepdims=True).reshape(1, 1, _C)
    q1 = jnp.sum(x1 * x1, axis=0, keepdims=True).reshape(1, 1, _C)
    s2 = jnp.sum(x2, axis=0, keepdims=True).reshape(1, 1, _C)
    q2 = jnp.sum(x2 * x2, axis=0, keepdims=True).reshape(1, 1, _C)

    @pl.when(h == 0)
    def _():
        s1_ref[...] = s1
        q1_ref[...] = q1
        s2_ref[...] = s2
        q2_ref[...] = q2

    @pl.when(h != 0)
    def _():
        s1_ref[...] += s1
        q1_ref[...] += q1
        s2_ref[...] += s2
        q2_ref[...] += q2


def _stats(x1t, x2t):
    out = jax.ShapeDtypeStruct((_B, 1, _C), jnp.float32)
    return pl.pallas_call(
        _stats_body,
        grid=(_B, _HW // _HWB),
        in_specs=[
            pl.BlockSpec((1, _HWB, _C), lambda b, h: (b, h, 0)),
            pl.BlockSpec((1, _HWB, _C), lambda b, h: (b, h, 0)),
        ],
        out_specs=[pl.BlockSpec((1, 1, _C), lambda b, h: (b, 0, 0))] * 4,
        out_shape=[out] * 4,
    )(x1t, x2t)


def _row_to_col(row):
    n = row.shape[1]
    ci = lax.broadcasted_iota(jnp.int32, (n, n), 0)
    cj = lax.broadcasted_iota(jnp.int32, (n, n), 1)
    return jnp.sum(jnp.where(ci == cj, row, 0.0), axis=1, keepdims=True)


def _select_body(s1_ref, q1_ref, s2_ref, q2_ref, g_ref, b_ref,
                 idx_ref, sc_ref, sh_ref):
    ci = lax.broadcasted_iota(jnp.int32, (_C2, _C2), 0)
    cj = lax.broadcasted_iota(jnp.int32, (_C2, _C2), 1)
    jrow = lax.broadcasted_iota(jnp.int32, (_C2, _C), 1)
    cval = lax.broadcasted_iota(jnp.int32, (_C2, _C), 0)

    def per_b(b, carry):
        ssum, qsum = carry
        vrow = jnp.concatenate(
            [s1_ref[pl.ds(b, 1), 0, :], s2_ref[pl.ds(b, 1), 0, :]], axis=1)
        qrow = jnp.concatenate(
            [q1_ref[pl.ds(b, 1), 0, :], q2_ref[pl.ds(b, 1), 0, :]], axis=1)
        vcol = _row_to_col(vrow)
        qcol = _row_to_col(qrow)
        beats = (vrow > vcol) | ((vrow == vcol) & (cj < ci))
        rank = jnp.sum(jnp.where(beats, 1, 0), axis=1, keepdims=True)
        iseq = rank == jrow
        onehot = jnp.where(iseq, 1.0, 0.0)
        ssum = ssum + jnp.sum(onehot * vcol, axis=0, keepdims=True)
        qsum = qsum + jnp.sum(onehot * qcol, axis=0, keepdims=True)
        idxrow = jnp.sum(jnp.where(iseq, cval, 0), axis=0, keepdims=True)
        idx_ref[pl.ds(b, 1), :] = idxrow
        return ssum, qsum

    zero = jnp.zeros((1, _C), jnp.float32)
    ssum, qsum = lax.fori_loop(0, _B, per_b, (zero, zero))
    mean = ssum * (1.0 / _N)
    var = qsum * (1.0 / _N) - mean * mean
    scale = g_ref[...] * lax.rsqrt(var + _EPS)
    sc_ref[...] = scale
    sh_ref[...] = b_ref[...] - mean * scale


def _select(s1, q1, s2, q2, gamma, beta):
    mat = pl.BlockSpec((_B, 1, _C), lambda: (0, 0, 0))
    vec = pl.BlockSpec((1, _C), lambda: (0, 0))
    return pl.pallas_call(
        _select_body,
        in_specs=[mat, mat, mat, mat, vec, vec],
        out_specs=[
            pl.BlockSpec((_B, _C), lambda: (0, 0)),
            vec,
            vec,
        ],
        out_shape=[
            jax.ShapeDtypeStruct((_B, _C), jnp.int32),
            jax.ShapeDtypeStruct((1, _C), jnp.float32),
            jax.ShapeDtypeStruct((1, _C), jnp.float32),
        ],
    )(s1, q1, s2, q2, gamma, beta)


def _gather_body(x1_ref, x2_ref, idx_ref, sc_ref, sh_ref, out_ref,
                 idx_v, roff_v, sc_v, sh_v, ibuf, obuf, g1sem, g2sem, ssem):
    cid = lax.axis_index("c")
    sid = lax.axis_index("s")
    b = sid
    hw0 = cid * _HWT

    def issue_in(k, slot):
        pltpu.async_copy(x1_ref.at[b, pl.ds(hw0 + k * _CH, _CH), :],
                         ibuf.at[pl.ds(slot * 2 * _CH, _CH), :],
                         g1sem.at[slot])
        pltpu.async_copy(x2_ref.at[b, pl.ds(hw0 + k * _CH, _CH), :],
                         ibuf.at[pl.ds(slot * 2 * _CH + _CH, _CH), :],
                         g2sem.at[slot])

    def compute_chunk(slot):
        for j in range(_NJ):
            colv = idx_v[pl.ds(j * _LANES, _LANES)]
            roffv = roff_v[pl.ds(j * _LANES, _LANES)]
            scv = sc_v[pl.ds(j * _LANES, _LANES)]
            shv = sh_v[pl.ds(j * _LANES, _LANES)]

            basev = jnp.full((_LANES,), slot * 2 * _CH, jnp.int32) + roffv

            @plsc.parallel_loop(0, _CH, unroll=_CH)
            def _(r, slot=slot, j=j, colv=colv, basev=basev, scv=scv,
                  shv=shv):
                g = plsc.load_gather(ibuf, [basev + r, colv])
                y = jnp.maximum(g * scv + shv, 0.0)
                obuf[slot, r, pl.ds(j * _LANES, _LANES)] = y

    for s in range(_NB):
        issue_in(s, s)
    pltpu.sync_copy(idx_ref.at[b], idx_v)
    pltpu.sync_copy(sc_ref.at[0], sc_v)
    pltpu.sync_copy(sh_ref.at[0], sh_v)
    for j in range(_NJ):
        iv = idx_v[pl.ds(j * _LANES, _LANES)]
        m = iv >= _C
        idx_v[pl.ds(j * _LANES, _LANES)] = jnp.where(m, iv - _C, iv)
        roff_v[pl.ds(j * _LANES, _LANES)] = jnp.where(m, _CH, 0)

    def outer(t, carry):
        for s in range(_NB):
            k = t * _NB + s
            pltpu.make_async_copy(x1_ref.at[0, pl.ds(0, _CH), :],
                                  ibuf.at[pl.ds(s * 2 * _CH, _CH), :],
                                  g1sem.at[s]).wait()
            pltpu.make_async_copy(x2_ref.at[0, pl.ds(0, _CH), :],
                                  ibuf.at[pl.ds(s * 2 * _CH + _CH, _CH), :],
                                  g2sem.at[s]).wait()

            @pl.when(t > 0)
            def _():
                pltpu.make_async_copy(obuf.at[s],
                                      out_ref.at[0, pl.ds(0, _CH), :],
                                      ssem.at[s]).wait()

            compute_chunk(s)
            pltpu.async_copy(obuf.at[s],
                             out_ref.at[b, pl.ds(hw0 + k * _CH, _CH), :],
                             ssem.at[s])

            @pl.when(k + _NB < _NCH)
            def _():
                issue_in(k + _NB, s)
        return carry

    lax.fori_loop(0, _NCH // _NB, outer, 0)

    for s in range(_NB):
        pltpu.make_async_copy(obuf.at[s], out_ref.at[0, pl.ds(0, _CH), :],
                              ssem.at[s]).wait()


def _gather(x1t, x2t, idx, scale, shift):
    mesh = plsc.VectorSubcoreMesh(core_axis_name="c", subcore_axis_name="s")
    fn = pl.kernel(
        _gather_body,
        out_type=jax.ShapeDtypeStruct((_B, _HW, _C), jnp.float32),
        mesh=mesh,
        compiler_params=pltpu.CompilerParams(needs_layout_passes=False),
        scratch_types=[
            pltpu.VMEM((_C,), jnp.int32),
            pltpu.VMEM((_C,), jnp.int32),
            pltpu.VMEM((_C,), jnp.float32),
            pltpu.VMEM((_C,), jnp.float32),
            pltpu.VMEM((_NB * 2 * _CH, _C), jnp.float32),
            pltpu.VMEM((_NB, _CH, _C), jnp.float32),
            pltpu.SemaphoreType.DMA((_NB,)),
            pltpu.SemaphoreType.DMA((_NB,)),
            pltpu.SemaphoreType.DMA((_NB,)),
        ],
    )
    return fn(x1t, x2t, idx, scale, shift)


def kernel(x1, x2, gamma, beta):
    x1t = x1.reshape(_B, _C, _HW).transpose(0, 2, 1)
    x2t = x2.reshape(_B, _C, _HW).transpose(0, 2, 1)
    s1, q1, s2, q2 = _stats(x1t, x2t)
    idx, scale, shift = _select(s1, q1, s2, q2,
                                gamma.reshape(1, _C), beta.reshape(1, _C))
    out = _gather(x1t, x2t, idx, scale, shift)
    return out.transpose(0, 2, 1).reshape(_B, _C, _H, _W)
